# baseline (speedup 1.0000x reference)
import functools

import jax
import jax.numpy as jnp
from jax import lax
from jax.experimental import pallas as pl
from jax.experimental.pallas import tpu as pltpu
from jax.experimental.pallas import tpu_sc as plsc

N = 10000
D = 128
DH = 64
DOUT = 64
E = 320000

NC = 2
NS = 16
NW = NC * NS

NPAD = 10240
CK = 128
EPW = 10240
EPAD = EPW * NW
NCHUNK = EPW // CK
RPT = NPAD // NS
RB = 1024
NBUF = 4
NHALF = 4
HC = NCHUNK // NHALF


def _mesh():
    return plsc.VectorSubcoreMesh(core_axis_name="c", subcore_axis_name="s")


@functools.cache
def _deg_kernel():
    @functools.partial(
        pl.kernel,
        out_type=jax.ShapeDtypeStruct((NC, NPAD), jnp.float32),
        mesh=_mesh(),
        scratch_types=[
            pltpu.VMEM_SHARED((NPAD,), jnp.float32),
            pltpu.VMEM((NCHUNK, CK), jnp.int32),
            pltpu.VMEM((CK,), jnp.float32),
            pltpu.VMEM((RPT,), jnp.float32),
        ],
    )
    def deg_k(cols3, out, dacc, colbuf, onesv, iov):
        c = lax.axis_index("c")
        s = lax.axis_index("s")
        wid = s * NC + c

        def fill16(i, _):
            onesv[pl.ds(i * 16, 16)] = jnp.ones((16,), jnp.float32)
            return 0

        lax.fori_loop(0, CK // 16, fill16, 0)

        def zfill(i, _):
            iov[pl.ds(i * 16, 16)] = jnp.zeros((16,), jnp.float32)
            return 0

        lax.fori_loop(0, RPT // 16, zfill, 0)
        pltpu.sync_copy(cols3.at[wid], colbuf)
        pltpu.sync_copy(iov, dacc.at[pl.ds(s * RPT, RPT)])
        plsc.subcore_barrier()

        def edge_step(j, _):
            pltpu.sync_copy(onesv, dacc.at[colbuf.at[j]], add=True)
            return 0

        lax.fori_loop(0, NCHUNK, edge_step, 0)
        plsc.subcore_barrier()

        pltpu.sync_copy(
            dacc.at[pl.ds(s * RPT, RPT)], out.at[c, pl.ds(s * RPT, RPT)]
        )

    return deg_k


@functools.cache
def _agg_kernel(nh):
    out_type = tuple(
        jax.ShapeDtypeStruct((NC, NPAD, DH), jnp.float32) for _ in range(nh)
    )

    @functools.partial(
        pl.kernel,
        out_type=out_type,
        mesh=_mesh(),
        compiler_params=pltpu.CompilerParams(use_tc_tiling_on_sc=False),
        scratch_types=[
            pltpu.VMEM_SHARED((NPAD, DH), jnp.float32),
            pltpu.VMEM_SHARED((NPAD, DH), jnp.float32),
            pltpu.VMEM((HC, CK), jnp.int32),
            pltpu.VMEM((HC, CK), jnp.int32),
        ]
        + [pltpu.VMEM((CK, DH), jnp.float32) for _ in range(NBUF)]
        + [pltpu.SemaphoreType.DMA for _ in range(NBUF)],
    )
    def agg_k(*refs):
        zs = refs[:nh]
        rows3, cols3 = refs[nh], refs[nh + 1]
        ps = refs[nh + 2 : 2 * nh + 2]
        zsh, acc, rowbuf, colbuf = refs[2 * nh + 2 : 2 * nh + 6]
        gbufs = refs[2 * nh + 6 : 2 * nh + 6 + NBUF]
        gsems = refs[2 * nh + 6 + NBUF :]

        c = lax.axis_index("c")
        s = lax.axis_index("s")
        wid = s * NC + c
        base_r = s * RPT

        for h in range(nh):
            def init_step(i, _):
                sl = pl.ds(base_r + i * CK, CK)
                pltpu.sync_copy(zs[h].at[sl], gbufs[0])
                pltpu.sync_copy(gbufs[0], zsh.at[sl])
                pltpu.sync_copy(gbufs[0], acc.at[sl])
                return 0

            lax.fori_loop(0, RPT // CK, init_step, 0)
            plsc.subcore_barrier()

            for eh in range(NHALF):
                pltpu.sync_copy(rows3.at[wid, pl.ds(eh * HC, HC)], rowbuf)
                pltpu.sync_copy(cols3.at[wid, pl.ds(eh * HC, HC)], colbuf)
                for b in range(NBUF):
                    pltpu.async_copy(zsh.at[rowbuf.at[b]], gbufs[b], gsems[b])

                def block_step(i, _):
                    for b in range(NBUF):
                        j = i * NBUF + b
                        pltpu.make_async_copy(
                            zsh.at[rowbuf.at[j]], gbufs[b], gsems[b]
                        ).wait()
                        pltpu.sync_copy(
                            gbufs[b], acc.at[colbuf.at[j]], add=True
                        )
                        jn = j + NBUF

                        @pl.when(jn < HC)
                        def _():
                            pltpu.async_copy(
                                zsh.at[rowbuf.at[jn]], gbufs[b], gsems[b]
                            )

                    return 0

                lax.fori_loop(0, HC // NBUF, block_step, 0)
            plsc.subcore_barrier()

            def out_step(i, _):
                sl = pl.ds(base_r + i * CK, CK)
                pltpu.sync_copy(acc.at[sl], gbufs[0])
                pltpu.sync_copy(gbufs[0], ps[h].at[c, sl])
                return 0

            lax.fori_loop(0, RPT // CK, out_step, 0)

    return agg_k


def _dinv_body(dg_ref, o_ref):
    o_ref[...] = lax.rsqrt(dg_ref[0] + dg_ref[1] + 1.0)


def _pre_body(x_ref, wlo_ref, whi_ref, dinv_ref, olo_ref, ohi_ref):
    xs = dinv_ref[...] * x_ref[...]
    olo_ref[...] = jnp.dot(xs, wlo_ref[...], preferred_element_type=jnp.float32)
    ohi_ref[...] = jnp.dot(xs, whi_ref[...], preferred_element_type=jnp.float32)


def _mid2_body(
    plo_ref, phi_ref, zlo_ref, zhi_ref, dinv_ref, blo_ref, bhi_ref,
    wlo_ref, whi_ref, olo_ref, ohi_ref,
):
    dv = dinv_ref[...]
    hl = jnp.maximum(dv * (plo_ref[0] + plo_ref[1] - zlo_ref[...]) + blo_ref[...], 0.0)
    hh = jnp.maximum(dv * (phi_ref[0] + phi_ref[1] - zhi_ref[...]) + bhi_ref[...], 0.0)
    h = dv * jnp.concatenate([hl, hh], axis=1)
    olo_ref[...] = jnp.dot(h, wlo_ref[...], preferred_element_type=jnp.float32)
    ohi_ref[...] = jnp.dot(h, whi_ref[...], preferred_element_type=jnp.float32)


def _mid1_body(
    plo_ref, phi_ref, zlo_ref, zhi_ref, dinv_ref, blo_ref, bhi_ref,
    w_ref, o_ref,
):
    dv = dinv_ref[...]
    hl = jnp.maximum(dv * (plo_ref[0] + plo_ref[1] - zlo_ref[...]) + blo_ref[...], 0.0)
    hh = jnp.maximum(dv * (phi_ref[0] + phi_ref[1] - zhi_ref[...]) + bhi_ref[...], 0.0)
    h = dv * jnp.concatenate([hl, hh], axis=1)
    o_ref[...] = jnp.dot(h, w_ref[...], preferred_element_type=jnp.float32)


def _fin_body(p_ref, z_ref, dinv_ref, b_ref, o_ref):
    t = dinv_ref[...] * (p_ref[0] + p_ref[1] - z_ref[...]) + b_ref[...]
    m = jnp.max(t, axis=1, keepdims=True)
    e = jnp.exp(t - m)
    o_ref[...] = t - m - jnp.log(jnp.sum(e, axis=1, keepdims=True))


def _rb(width):
    return pl.BlockSpec((RB, width), lambda i: (i, 0))


def _pb(width):
    return pl.BlockSpec((NC, RB, width), lambda i: (0, i, 0))


def _cb(r, cdim):
    return pl.BlockSpec((r, cdim), lambda i: (0, 0))


def _pre(x_pad, Wlo, Whi, dinv_col):
    return pl.pallas_call(
        _pre_body,
        grid=(NPAD // RB,),
        in_specs=[_rb(D), _cb(D, DH), _cb(D, DH), _rb(1)],
        out_specs=[_rb(DH), _rb(DH)],
        out_shape=[
            jax.ShapeDtypeStruct((NPAD, DH), jnp.float32),
            jax.ShapeDtypeStruct((NPAD, DH), jnp.float32),
        ],
    )(x_pad, Wlo, Whi, dinv_col)


def _mid2(plo, phi, zlo, zhi, dinv_col, b, W):
    return pl.pallas_call(
        _mid2_body,
        grid=(NPAD // RB,),
        in_specs=[
            _pb(DH), _pb(DH), _rb(DH), _rb(DH), _rb(1),
            _cb(1, DH), _cb(1, DH), _cb(D, DH), _cb(D, DH),
        ],
        out_specs=[_rb(DH), _rb(DH)],
        out_shape=[
            jax.ShapeDtypeStruct((NPAD, DH), jnp.float32),
            jax.ShapeDtypeStruct((NPAD, DH), jnp.float32),
        ],
    )(
        plo, phi, zlo, zhi, dinv_col,
        b[:DH].reshape(1, DH), b[DH:].reshape(1, DH),
        W[:, :DH], W[:, DH:],
    )


def _mid1(plo, phi, zlo, zhi, dinv_col, b, W):
    return pl.pallas_call(
        _mid1_body,
        grid=(NPAD // RB,),
        in_specs=[
            _pb(DH), _pb(DH), _rb(DH), _rb(DH), _rb(1),
            _cb(1, DH), _cb(1, DH), _cb(D, DOUT),
        ],
        out_specs=_rb(DOUT),
        out_shape=jax.ShapeDtypeStruct((NPAD, DOUT), jnp.float32),
    )(
        plo, phi, zlo, zhi, dinv_col,
        b[:DH].reshape(1, DH), b[DH:].reshape(1, DH), W,
    )


def _fin(p, zs_prev, dinv_col, b):
    return pl.pallas_call(
        _fin_body,
        grid=(NPAD // RB,),
        in_specs=[_pb(DOUT), _rb(DOUT), _rb(1), _cb(1, DOUT)],
        out_specs=_rb(DOUT),
        out_shape=jax.ShapeDtypeStruct((NPAD, DOUT), jnp.float32),
    )(p, zs_prev, dinv_col, b.reshape(1, DOUT))


def _deg_sc(cols_p):
    return _deg_kernel()(cols_p)


def _agg_sc2(zlo, zhi, rows_p, cols_p):
    return _agg_kernel(2)(zlo, zhi, rows_p, cols_p)


def _agg_sc1(zs, rows_p, cols_p):
    return _agg_kernel(1)(zs, rows_p, cols_p)[0]


def kernel(x, edge_index, W1, b1, W2, b2, W3, b3):
    pad_e = EPAD - E
    rows_p = jnp.concatenate(
        [edge_index[0], jnp.zeros((pad_e,), jnp.int32)]
    ).reshape(NW, NCHUNK, CK)
    cols_p = jnp.concatenate(
        [edge_index[1], jnp.full((pad_e,), N, jnp.int32)]
    ).reshape(NW, NCHUNK, CK)
    x_pad = jnp.pad(x, ((0, NPAD - N), (0, 0)))

    deg = _deg_sc(cols_p)
    dinv80 = pl.pallas_call(
        _dinv_body,
        out_shape=jax.ShapeDtypeStruct((NPAD // 128, 128), jnp.float32),
    )(deg.reshape(NC, NPAD // 128, 128))
    dinv_col = dinv80.reshape(NPAD, 1)

    zs1lo, zs1hi = _pre(x_pad, W1[:, :DH], W1[:, DH:], dinv_col)
    p1lo, p1hi = _agg_sc2(zs1lo, zs1hi, rows_p, cols_p)
    zs2lo, zs2hi = _mid2(p1lo, p1hi, zs1lo, zs1hi, dinv_col, b1, W2)
    p2lo, p2hi = _agg_sc2(zs2lo, zs2hi, rows_p, cols_p)
    zs3 = _mid1(p2lo, p2hi, zs2lo, zs2hi, dinv_col, b2, W3)
    p3 = _agg_sc1(zs3, rows_p, cols_p)
    logits = _fin(p3, zs3, dinv_col, b3)
    return logits[:N]


# speedup vs baseline: 1.1985x; 1.1985x over previous
import functools

import jax
import jax.numpy as jnp
from jax import lax
from jax.experimental import pallas as pl
from jax.experimental.pallas import tpu as pltpu
from jax.experimental.pallas import tpu_sc as plsc

N = 10000
D = 128
DH = 64
DOUT = 64
E = 320000

NC = 2
NS = 16
NW = NC * NS

NPAD = 10240
CK = 128
EPW = 10240
EPAD = EPW * NW
NCHUNK = EPW // CK
RPT = NPAD // NS
RB = 1024
NBUF = 2
NHALF = 2
HC = NCHUNK // NHALF


def _mesh():
    return plsc.VectorSubcoreMesh(core_axis_name="c", subcore_axis_name="s")


@functools.cache
def _deg_kernel():
    @functools.partial(
        pl.kernel,
        out_type=jax.ShapeDtypeStruct((NC, NPAD), jnp.float32),
        mesh=_mesh(),
        scratch_types=[
            pltpu.VMEM_SHARED((NPAD,), jnp.float32),
            pltpu.VMEM((NCHUNK, CK), jnp.int32),
            pltpu.VMEM((CK,), jnp.float32),
            pltpu.VMEM((RPT,), jnp.float32),
        ],
    )
    def deg_k(cols3, out, dacc, colbuf, onesv, iov):
        c = lax.axis_index("c")
        s = lax.axis_index("s")
        wid = s * NC + c

        def fill16(i, _):
            onesv[pl.ds(i * 16, 16)] = jnp.ones((16,), jnp.float32)
            return 0

        lax.fori_loop(0, CK // 16, fill16, 0)

        def zfill(i, _):
            iov[pl.ds(i * 16, 16)] = jnp.zeros((16,), jnp.float32)
            return 0

        lax.fori_loop(0, RPT // 16, zfill, 0)
        pltpu.sync_copy(cols3.at[wid], colbuf)
        pltpu.sync_copy(iov, dacc.at[pl.ds(s * RPT, RPT)])
        plsc.subcore_barrier()

        def edge_step(j, _):
            pltpu.sync_copy(onesv, dacc.at[colbuf.at[j]], add=True)
            return 0

        lax.fori_loop(0, NCHUNK, edge_step, 0)
        plsc.subcore_barrier()

        pltpu.sync_copy(
            dacc.at[pl.ds(s * RPT, RPT)], out.at[c, pl.ds(s * RPT, RPT)]
        )

    return deg_k


@functools.cache
def _agg_kernel(nh):
    @functools.partial(
        pl.kernel,
        out_type=jax.ShapeDtypeStruct((NC, NPAD, D), jnp.float32),
        mesh=_mesh(),
        compiler_params=pltpu.CompilerParams(use_tc_tiling_on_sc=False),
        scratch_types=[
            pltpu.VMEM_SHARED((NPAD, DH), jnp.float32),
            pltpu.VMEM_SHARED((NPAD, DH), jnp.float32),
            pltpu.VMEM((HC, CK), jnp.int32),
            pltpu.VMEM((HC, CK), jnp.int32),
        ]
        + [pltpu.VMEM((CK, DH), jnp.float32) for _ in range(NBUF)]
        + [pltpu.SemaphoreType.DMA for _ in range(NBUF)],
    )
    def agg_k(zs, rows3, cols3, out, *refs):
        zsh, acc, rowbuf, colbuf = refs[:4]
        gbufs = refs[4 : 4 + NBUF]
        gsems = refs[4 + NBUF :]

        c = lax.axis_index("c")
        s = lax.axis_index("s")
        wid = s * NC + c
        base_r = s * RPT

        for h in range(nh):
            def init_step(i, _):
                sl = pl.ds(base_r + i * CK, CK)
                pltpu.sync_copy(zs.at[sl, pl.ds(h * DH, DH)], gbufs[0])
                pltpu.sync_copy(gbufs[0], zsh.at[sl])
                pltpu.sync_copy(gbufs[0], acc.at[sl])
                return 0

            lax.fori_loop(0, RPT // CK, init_step, 0)
            plsc.subcore_barrier()

            for eh in range(NHALF):
                pltpu.sync_copy(rows3.at[wid, pl.ds(eh * HC, HC)], rowbuf)
                pltpu.sync_copy(cols3.at[wid, pl.ds(eh * HC, HC)], colbuf)
                for b in range(NBUF):
                    pltpu.async_copy(zsh.at[rowbuf.at[b]], gbufs[b], gsems[b])

                def block_step(i, _):
                    for b in range(NBUF):
                        j = i * NBUF + b
                        pltpu.make_async_copy(
                            zsh.at[rowbuf.at[j]], gbufs[b], gsems[b]
                        ).wait()
                        pltpu.sync_copy(
                            gbufs[b], acc.at[colbuf.at[j]], add=True
                        )
                        jn = j + NBUF

                        @pl.when(jn < HC)
                        def _():
                            pltpu.async_copy(
                                zsh.at[rowbuf.at[jn]], gbufs[b], gsems[b]
                            )

                    return 0

                lax.fori_loop(0, HC // NBUF, block_step, 0)
            plsc.subcore_barrier()

            def out_step(i, _):
                sl = pl.ds(base_r + i * CK, CK)
                pltpu.sync_copy(acc.at[sl], gbufs[0])
                pltpu.sync_copy(gbufs[0], out.at[c, sl, pl.ds(h * DH, DH)])
                return 0

            lax.fori_loop(0, RPT // CK, out_step, 0)

    return agg_k


def _dinv_body(dg_ref, o_ref):
    o_ref[...] = lax.rsqrt(dg_ref[0] + dg_ref[1] + 1.0)


def _pre_body(x_ref, w_ref, dinv_ref, o_ref):
    o_ref[...] = jnp.dot(
        dinv_ref[...] * x_ref[...], w_ref[...],
        preferred_element_type=jnp.float32,
    )


def _mid_body(p_ref, z_ref, dinv_ref, b_ref, w_ref, o_ref):
    dv = dinv_ref[...]
    h = jnp.maximum(dv * (p_ref[0] + p_ref[1] - z_ref[...]) + b_ref[...], 0.0)
    o_ref[...] = jnp.dot(dv * h, w_ref[...], preferred_element_type=jnp.float32)


def _fin_body(p_ref, z_ref, dinv_ref, b_ref, o_ref):
    t = dinv_ref[...] * (p_ref[0] + p_ref[1] - z_ref[...]) + b_ref[...]
    valid = lax.broadcasted_iota(jnp.int32, t.shape, 1) < DOUT
    tm = jnp.where(valid, t, -jnp.inf)
    m = jnp.max(tm, axis=1, keepdims=True)
    e = jnp.where(valid, jnp.exp(t - m), 0.0)
    ls = t - m - jnp.log(jnp.sum(e, axis=1, keepdims=True))
    o_ref[...] = ls[:, :DOUT]


def _rb(width):
    return pl.BlockSpec((RB, width), lambda i: (i, 0))


def _cb(r, cdim):
    return pl.BlockSpec((r, cdim), lambda i: (0, 0))


def _pre(x_pad, W, dinv_col):
    return pl.pallas_call(
        _pre_body,
        grid=(NPAD // RB,),
        in_specs=[_rb(D), _cb(D, D), _rb(1)],
        out_specs=_rb(D),
        out_shape=jax.ShapeDtypeStruct((NPAD, D), jnp.float32),
    )(x_pad, W, dinv_col)


def _mid(p, zs_prev, dinv_col, b, W):
    return pl.pallas_call(
        _mid_body,
        grid=(NPAD // RB,),
        in_specs=[
            pl.BlockSpec((NC, RB, D), lambda i: (0, i, 0)),
            _rb(D), _rb(1), _cb(1, D), _cb(D, D),
        ],
        out_specs=_rb(D),
        out_shape=jax.ShapeDtypeStruct((NPAD, D), jnp.float32),
    )(p, zs_prev, dinv_col, b.reshape(1, D), W)


def _fin(p, zs_prev, dinv_col, b_pad):
    return pl.pallas_call(
        _fin_body,
        grid=(NPAD // RB,),
        in_specs=[
            pl.BlockSpec((NC, RB, D), lambda i: (0, i, 0)),
            _rb(D), _rb(1), _cb(1, D),
        ],
        out_specs=_rb(DOUT),
        out_shape=jax.ShapeDtypeStruct((NPAD, DOUT), jnp.float32),
    )(p, zs_prev, dinv_col, b_pad.reshape(1, D))


def _deg_sc(cols_p):
    return _deg_kernel()(cols_p)


def _agg_sc2(zs, rows_p, cols_p):
    return _agg_kernel(2)(zs, rows_p, cols_p)


def _agg_sc1(zs, rows_p, cols_p):
    return _agg_kernel(1)(zs, rows_p, cols_p)


def kernel(x, edge_index, W1, b1, W2, b2, W3, b3):
    pad_e = EPAD - E
    rows_p = jnp.concatenate(
        [edge_index[0], jnp.zeros((pad_e,), jnp.int32)]
    ).reshape(NW, NCHUNK, CK)
    cols_p = jnp.concatenate(
        [edge_index[1], jnp.full((pad_e,), N, jnp.int32)]
    ).reshape(NW, NCHUNK, CK)
    x_pad = jnp.pad(x, ((0, NPAD - N), (0, 0)))

    deg = _deg_sc(cols_p)
    dinv80 = pl.pallas_call(
        _dinv_body,
        out_shape=jax.ShapeDtypeStruct((NPAD // 128, 128), jnp.float32),
    )(deg.reshape(NC, NPAD // 128, 128))
    dinv_col = dinv80.reshape(NPAD, 1)

    zs1 = _pre(x_pad, W1, dinv_col)
    p1 = _agg_sc2(zs1, rows_p, cols_p)
    zs2 = _mid(p1, zs1, dinv_col, b1, W2)
    p2 = _agg_sc2(zs2, rows_p, cols_p)
    W3p = jnp.pad(W3, ((0, 0), (0, D - DOUT)))
    b3p = jnp.pad(b3, ((0, D - DOUT),))
    zs3 = _mid(p2, zs2, dinv_col, b2, W3p)
    p3 = _agg_sc1(zs3, rows_p, cols_p)
    logits = _fin(p3, zs3, dinv_col, b3p)
    return logits[:N]


# speedup vs baseline: 1.2003x; 1.0015x over previous
import functools

import jax
import jax.numpy as jnp
from jax import lax
from jax.experimental import pallas as pl
from jax.experimental.pallas import tpu as pltpu
from jax.experimental.pallas import tpu_sc as plsc

N = 10000
D = 128
DH = 64
DOUT = 64
E = 320000

NC = 2
NS = 16
NW = NC * NS

NPAD = 10240
CK = 128
EPW = 10240
EPAD = EPW * NW
NCHUNK = EPW // CK
RPT = NPAD // NS
RB = 1024
NBUF = 2
NHALF = 2
HC = NCHUNK // NHALF


def _mesh():
    return plsc.VectorSubcoreMesh(core_axis_name="c", subcore_axis_name="s")


@functools.cache
def _deg_kernel():
    @functools.partial(
        pl.kernel,
        out_type=jax.ShapeDtypeStruct((NC, NPAD), jnp.float32),
        mesh=_mesh(),
        scratch_types=[
            pltpu.VMEM_SHARED((NPAD,), jnp.float32),
            pltpu.VMEM((NCHUNK, CK), jnp.int32),
            pltpu.VMEM((CK,), jnp.float32),
            pltpu.VMEM((RPT,), jnp.float32),
        ],
    )
    def deg_k(cols3, out, dacc, colbuf, onesv, iov):
        c = lax.axis_index("c")
        s = lax.axis_index("s")
        wid = s * NC + c

        def fill16(i, _):
            onesv[pl.ds(i * 16, 16)] = jnp.ones((16,), jnp.float32)
            return 0

        lax.fori_loop(0, CK // 16, fill16, 0)

        def zfill(i, _):
            iov[pl.ds(i * 16, 16)] = jnp.zeros((16,), jnp.float32)
            return 0

        lax.fori_loop(0, RPT // 16, zfill, 0)
        pltpu.sync_copy(cols3.at[wid], colbuf)
        pltpu.sync_copy(iov, dacc.at[pl.ds(s * RPT, RPT)])
        plsc.subcore_barrier()

        def edge_step(j, _):
            pltpu.sync_copy(onesv, dacc.at[colbuf.at[j]], add=True)
            return 0

        lax.fori_loop(0, NCHUNK, edge_step, 0)
        plsc.subcore_barrier()

        pltpu.sync_copy(
            dacc.at[pl.ds(s * RPT, RPT)], out.at[c, pl.ds(s * RPT, RPT)]
        )

    return deg_k


@functools.cache
def _agg_kernel(nh):
    @functools.partial(
        pl.kernel,
        out_type=jax.ShapeDtypeStruct((NC, NPAD, D), jnp.float32),
        mesh=_mesh(),
        compiler_params=pltpu.CompilerParams(use_tc_tiling_on_sc=False),
        scratch_types=[
            pltpu.VMEM_SHARED((NPAD, DH), jnp.float32),
            pltpu.VMEM_SHARED((NPAD, DH), jnp.float32),
            pltpu.VMEM((HC, CK), jnp.int32),
            pltpu.VMEM((HC, CK), jnp.int32),
        ]
        + [pltpu.VMEM((CK, DH), jnp.float32) for _ in range(NBUF)]
        + [pltpu.SemaphoreType.DMA for _ in range(NBUF)],
    )
    def agg_k(zs, rows3, cols3, out, *refs):
        zsh, acc, rowbuf, colbuf = refs[:4]
        gbufs = refs[4 : 4 + NBUF]
        gsems = refs[4 + NBUF :]

        c = lax.axis_index("c")
        s = lax.axis_index("s")
        wid = s * NC + c
        base_r = s * RPT

        for h in range(nh):
            def init_step(i, _):
                sl = pl.ds(base_r + i * CK, CK)
                pltpu.sync_copy(zs.at[sl, pl.ds(h * DH, DH)], gbufs[0])
                pltpu.sync_copy(gbufs[0], zsh.at[sl])
                pltpu.sync_copy(gbufs[0], acc.at[sl])
                return 0

            lax.fori_loop(0, RPT // CK, init_step, 0)
            plsc.subcore_barrier()

            for eh in range(NHALF):
                pltpu.sync_copy(rows3.at[wid, pl.ds(eh * HC, HC)], rowbuf)
                pltpu.sync_copy(cols3.at[wid, pl.ds(eh * HC, HC)], colbuf)
                for b in range(NBUF):
                    pltpu.async_copy(zsh.at[rowbuf.at[b]], gbufs[b], gsems[b])

                def block_step(i, _):
                    for b in range(NBUF):
                        j = i * NBUF + b
                        pltpu.make_async_copy(
                            zsh.at[rowbuf.at[j]], gbufs[b], gsems[b]
                        ).wait()
                        pltpu.sync_copy(
                            gbufs[b], acc.at[colbuf.at[j]], add=True
                        )
                        jn = j + NBUF

                        @pl.when(jn < HC)
                        def _():
                            pltpu.async_copy(
                                zsh.at[rowbuf.at[jn]], gbufs[b], gsems[b]
                            )

                    return 0

                lax.fori_loop(0, HC // NBUF, block_step, 0)
            plsc.subcore_barrier()

            def out_step(i, _):
                sl = pl.ds(base_r + i * CK, CK)
                pltpu.sync_copy(acc.at[sl], gbufs[0])
                pltpu.sync_copy(gbufs[0], out.at[c, sl, pl.ds(h * DH, DH)])
                return 0

            lax.fori_loop(0, RPT // CK, out_step, 0)

    return agg_k


def _dinv_body(dg_ref, o_ref):
    o_ref[...] = lax.rsqrt(dg_ref[0] + dg_ref[1] + 1.0)


def _pre_body(x_ref, w_ref, dinv_ref, o_ref):
    xs = (dinv_ref[...] * x_ref[...]).astype(jnp.bfloat16)
    o_ref[...] = jnp.dot(
        xs, w_ref[...].astype(jnp.bfloat16),
        preferred_element_type=jnp.float32,
    )


def _mid_body(p_ref, z_ref, dinv_ref, b_ref, w_ref, o_ref):
    dv = dinv_ref[...]
    h = jnp.maximum(dv * (p_ref[0] + p_ref[1] - z_ref[...]) + b_ref[...], 0.0)
    o_ref[...] = jnp.dot(
        (dv * h).astype(jnp.bfloat16), w_ref[...].astype(jnp.bfloat16),
        preferred_element_type=jnp.float32,
    )


def _fin_body(p_ref, z_ref, dinv_ref, b_ref, o_ref):
    t = dinv_ref[...] * (p_ref[0] + p_ref[1] - z_ref[...]) + b_ref[...]
    valid = lax.broadcasted_iota(jnp.int32, t.shape, 1) < DOUT
    tm = jnp.where(valid, t, -jnp.inf)
    m = jnp.max(tm, axis=1, keepdims=True)
    e = jnp.where(valid, jnp.exp(t - m), 0.0)
    ls = t - m - jnp.log(jnp.sum(e, axis=1, keepdims=True))
    o_ref[...] = ls[:, :DOUT]


def _rb(width):
    return pl.BlockSpec((RB, width), lambda i: (i, 0))


def _cb(r, cdim):
    return pl.BlockSpec((r, cdim), lambda i: (0, 0))


def _pre(x_pad, W, dinv_col):
    return pl.pallas_call(
        _pre_body,
        grid=(NPAD // RB,),
        in_specs=[_rb(D), _cb(D, D), _rb(1)],
        out_specs=_rb(D),
        out_shape=jax.ShapeDtypeStruct((NPAD, D), jnp.float32),
    )(x_pad, W, dinv_col)


def _mid(p, zs_prev, dinv_col, b, W):
    return pl.pallas_call(
        _mid_body,
        grid=(NPAD // RB,),
        in_specs=[
            pl.BlockSpec((NC, RB, D), lambda i: (0, i, 0)),
            _rb(D), _rb(1), _cb(1, D), _cb(D, D),
        ],
        out_specs=_rb(D),
        out_shape=jax.ShapeDtypeStruct((NPAD, D), jnp.float32),
    )(p, zs_prev, dinv_col, b.reshape(1, D), W)


def _fin(p, zs_prev, dinv_col, b_pad):
    return pl.pallas_call(
        _fin_body,
        grid=(NPAD // RB,),
        in_specs=[
            pl.BlockSpec((NC, RB, D), lambda i: (0, i, 0)),
            _rb(D), _rb(1), _cb(1, D),
        ],
        out_specs=_rb(DOUT),
        out_shape=jax.ShapeDtypeStruct((NPAD, DOUT), jnp.float32),
    )(p, zs_prev, dinv_col, b_pad.reshape(1, D))


def _deg_sc(cols_p):
    return _deg_kernel()(cols_p)


def _agg_sc2(zs, rows_p, cols_p):
    return _agg_kernel(2)(zs, rows_p, cols_p)


def _agg_sc1(zs, rows_p, cols_p):
    return _agg_kernel(1)(zs, rows_p, cols_p)


def kernel(x, edge_index, W1, b1, W2, b2, W3, b3):
    pad_e = EPAD - E
    rows_p = jnp.concatenate(
        [edge_index[0], jnp.zeros((pad_e,), jnp.int32)]
    ).reshape(NW, NCHUNK, CK)
    cols_p = jnp.concatenate(
        [edge_index[1], jnp.full((pad_e,), N, jnp.int32)]
    ).reshape(NW, NCHUNK, CK)
    x_pad = jnp.pad(x, ((0, NPAD - N), (0, 0)))

    deg = _deg_sc(cols_p)
    dinv80 = pl.pallas_call(
        _dinv_body,
        out_shape=jax.ShapeDtypeStruct((NPAD // 128, 128), jnp.float32),
    )(deg.reshape(NC, NPAD // 128, 128))
    dinv_col = dinv80.reshape(NPAD, 1)

    zs1 = _pre(x_pad, W1, dinv_col)
    p1 = _agg_sc2(zs1, rows_p, cols_p)
    zs2 = _mid(p1, zs1, dinv_col, b1, W2)
    p2 = _agg_sc2(zs2, rows_p, cols_p)
    W3p = jnp.pad(W3, ((0, 0), (0, D - DOUT)))
    b3p = jnp.pad(b3, ((0, D - DOUT),))
    zs3 = _mid(p2, zs2, dinv_col, b2, W3p)
    p3 = _agg_sc1(zs3, rows_p, cols_p)
    logits = _fin(p3, zs3, dinv_col, b3p)
    return logits[:N]


# speedup vs baseline: 1.2016x; 1.0010x over previous
import functools

import jax
import jax.numpy as jnp
from jax import lax
from jax.experimental import pallas as pl
from jax.experimental.pallas import tpu as pltpu
from jax.experimental.pallas import tpu_sc as plsc

N = 10000
D = 128
DH = 64
DOUT = 64
E = 320000

NC = 2
NS = 16
NW = NC * NS

NPAD = 10240
CK = 128
EPW = 10240
EPAD = EPW * NW
NCHUNK = EPW // CK
RPT = NPAD // NS
RB = 1024
NBUF = 2
NHALF = 2
HC = NCHUNK // NHALF


def _mesh():
    return plsc.VectorSubcoreMesh(core_axis_name="c", subcore_axis_name="s")


@functools.cache
def _deg_kernel():
    @functools.partial(
        pl.kernel,
        out_type=jax.ShapeDtypeStruct((NC, NPAD), jnp.float32),
        mesh=_mesh(),
        scratch_types=[
            pltpu.VMEM_SHARED((NPAD,), jnp.float32),
            pltpu.VMEM((NCHUNK, CK), jnp.int32),
            pltpu.VMEM((CK,), jnp.float32),
            pltpu.VMEM((RPT,), jnp.float32),
        ],
    )
    def deg_k(cols3, out, dacc, colbuf, onesv, iov):
        c = lax.axis_index("c")
        s = lax.axis_index("s")
        wid = s * NC + c

        def fill16(i, _):
            onesv[pl.ds(i * 16, 16)] = jnp.ones((16,), jnp.float32)
            return 0

        lax.fori_loop(0, CK // 16, fill16, 0)

        def zfill(i, _):
            iov[pl.ds(i * 16, 16)] = jnp.zeros((16,), jnp.float32)
            return 0

        lax.fori_loop(0, RPT // 16, zfill, 0)
        pltpu.sync_copy(cols3.at[wid], colbuf)
        pltpu.sync_copy(iov, dacc.at[pl.ds(s * RPT, RPT)])
        plsc.subcore_barrier()

        def edge_step(j, _):
            pltpu.sync_copy(onesv, dacc.at[colbuf.at[j]], add=True)
            return 0

        lax.fori_loop(0, NCHUNK, edge_step, 0)
        plsc.subcore_barrier()

        pltpu.sync_copy(
            dacc.at[pl.ds(s * RPT, RPT)], out.at[c, pl.ds(s * RPT, RPT)]
        )

    return deg_k


@functools.cache
def _agg_kernel(nh):
    @functools.partial(
        pl.kernel,
        out_type=jax.ShapeDtypeStruct((NC, NPAD, D), jnp.float32),
        mesh=_mesh(),
        compiler_params=pltpu.CompilerParams(use_tc_tiling_on_sc=False),
        scratch_types=[
            pltpu.VMEM_SHARED((NPAD, DH), jnp.float32),
            pltpu.VMEM_SHARED((NPAD, DH), jnp.float32),
            pltpu.VMEM((HC, CK), jnp.int32),
            pltpu.VMEM((HC, CK), jnp.int32),
        ]
        + [pltpu.VMEM((CK, DH), jnp.float32) for _ in range(NBUF)]
        + [pltpu.SemaphoreType.DMA for _ in range(NBUF)],
    )
    def agg_k(zs, rows3, cols3, out, *refs):
        zsh, acc, rowbuf, colbuf = refs[:4]
        gbufs = refs[4 : 4 + NBUF]
        gsems = refs[4 + NBUF :]

        c = lax.axis_index("c")
        s = lax.axis_index("s")
        wid = s * NC + c
        base_r = s * RPT

        for h in range(nh):
            def init_step(i, _):
                sl = pl.ds(base_r + i * CK, CK)
                pltpu.sync_copy(zs.at[sl, pl.ds(h * DH, DH)], gbufs[0])
                pltpu.sync_copy(gbufs[0], zsh.at[sl])
                pltpu.sync_copy(gbufs[0], acc.at[sl])
                return 0

            lax.fori_loop(0, RPT // CK, init_step, 0)
            plsc.subcore_barrier()

            for eh in range(NHALF):
                pltpu.sync_copy(rows3.at[wid, pl.ds(eh * HC, HC)], rowbuf)
                pltpu.sync_copy(cols3.at[wid, pl.ds(eh * HC, HC)], colbuf)
                for b in range(NBUF):
                    pltpu.async_copy(zsh.at[rowbuf.at[b]], gbufs[b], gsems[b])

                def block_step(i, _):
                    for b in range(NBUF):
                        j = i * NBUF + b
                        pltpu.make_async_copy(
                            zsh.at[rowbuf.at[j]], gbufs[b], gsems[b]
                        ).wait()
                        pltpu.sync_copy(
                            gbufs[b], acc.at[colbuf.at[j]], add=True
                        )
                        jn = j + NBUF

                        @pl.when(jn < HC)
                        def _():
                            pltpu.async_copy(
                                zsh.at[rowbuf.at[jn]], gbufs[b], gsems[b]
                            )

                    return 0

                lax.fori_loop(0, HC // NBUF, block_step, 0)
            plsc.subcore_barrier()

            def out_step(i, _):
                sl = pl.ds(base_r + i * CK, CK)
                pltpu.sync_copy(acc.at[sl], gbufs[0])
                pltpu.sync_copy(gbufs[0], out.at[c, sl, pl.ds(h * DH, DH)])
                return 0

            lax.fori_loop(0, RPT // CK, out_step, 0)

    return agg_k


def _dinv_body(dg_ref, o_ref):
    o_ref[...] = lax.rsqrt(dg_ref[0] + dg_ref[1] + 1.0)


def _pre_body(x_ref, w_ref, dinv_ref, o_ref):
    o_ref[...] = jnp.dot(
        dinv_ref[...] * x_ref[...], w_ref[...],
        preferred_element_type=jnp.float32,
    )


def _mid_body(p_ref, z_ref, dinv_ref, b_ref, w_ref, o_ref):
    dv = dinv_ref[...]
    h = jnp.maximum(dv * (p_ref[0] + p_ref[1] - z_ref[...]) + b_ref[...], 0.0)
    o_ref[...] = jnp.dot(dv * h, w_ref[...], preferred_element_type=jnp.float32)


def _fin_body(p_ref, z_ref, dinv_ref, b_ref, o_ref):
    t = dinv_ref[...] * (p_ref[0] + p_ref[1] - z_ref[...]) + b_ref[...]
    valid = lax.broadcasted_iota(jnp.int32, t.shape, 1) < DOUT
    tm = jnp.where(valid, t, -jnp.inf)
    m = jnp.max(tm, axis=1, keepdims=True)
    e = jnp.where(valid, jnp.exp(t - m), 0.0)
    ls = t - m - jnp.log(jnp.sum(e, axis=1, keepdims=True))
    o_ref[...] = ls[:, :DOUT]


def _rb(width):
    return pl.BlockSpec((RB, width), lambda i: (i, 0))


def _cb(r, cdim):
    return pl.BlockSpec((r, cdim), lambda i: (0, 0))


def _pre(x_pad, W, dinv_col):
    return pl.pallas_call(
        _pre_body,
        grid=(NPAD // RB,),
        in_specs=[_rb(D), _cb(D, D), _rb(1)],
        out_specs=_rb(D),
        out_shape=jax.ShapeDtypeStruct((NPAD, D), jnp.float32),
    )(x_pad, W, dinv_col)


def _mid(p, zs_prev, dinv_col, b, W):
    return pl.pallas_call(
        _mid_body,
        grid=(NPAD // RB,),
        in_specs=[
            pl.BlockSpec((NC, RB, D), lambda i: (0, i, 0)),
            _rb(D), _rb(1), _cb(1, D), _cb(D, D),
        ],
        out_specs=_rb(D),
        out_shape=jax.ShapeDtypeStruct((NPAD, D), jnp.float32),
    )(p, zs_prev, dinv_col, b.reshape(1, D), W)


def _fin(p, zs_prev, dinv_col, b_pad):
    return pl.pallas_call(
        _fin_body,
        grid=(NPAD // RB,),
        in_specs=[
            pl.BlockSpec((NC, RB, D), lambda i: (0, i, 0)),
            _rb(D), _rb(1), _cb(1, D),
        ],
        out_specs=_rb(DOUT),
        out_shape=jax.ShapeDtypeStruct((NPAD, DOUT), jnp.float32),
    )(p, zs_prev, dinv_col, b_pad.reshape(1, D))


def _deg_sc(cols_p):
    return _deg_kernel()(cols_p)


def _agg_sc2(zs, rows_p, cols_p):
    return _agg_kernel(2)(zs, rows_p, cols_p)


def _agg_sc1(zs, rows_p, cols_p):
    return _agg_kernel(1)(zs, rows_p, cols_p)


def kernel(x, edge_index, W1, b1, W2, b2, W3, b3):
    pad_e = EPAD - E
    rows_p = jnp.concatenate(
        [edge_index[0], jnp.zeros((pad_e,), jnp.int32)]
    ).reshape(NW, NCHUNK, CK)
    cols_p = jnp.concatenate(
        [edge_index[1], jnp.full((pad_e,), N, jnp.int32)]
    ).reshape(NW, NCHUNK, CK)
    x_pad = jnp.pad(x, ((0, NPAD - N), (0, 0)))

    deg = _deg_sc(cols_p)
    dinv80 = pl.pallas_call(
        _dinv_body,
        out_shape=jax.ShapeDtypeStruct((NPAD // 128, 128), jnp.float32),
    )(deg.reshape(NC, NPAD // 128, 128))
    dinv_col = dinv80.reshape(NPAD, 1)

    zs1 = _pre(x_pad, W1, dinv_col)
    p1 = _agg_sc2(zs1, rows_p, cols_p)
    zs2 = _mid(p1, zs1, dinv_col, b1, W2)
    p2 = _agg_sc2(zs2, rows_p, cols_p)
    W3p = jnp.pad(W3, ((0, 0), (0, D - DOUT)))
    b3p = jnp.pad(b3, ((0, D - DOUT),))
    zs3 = _mid(p2, zs2, dinv_col, b2, W3p)
    p3 = _agg_sc1(zs3, rows_p, cols_p)
    logits = _fin(p3, zs3, dinv_col, b3p)
    return logits[:N]
